# Initial kernel scaffold; baseline (speedup 1.0000x reference)
import functools

import jax
import jax.numpy as jnp
from jax import lax
from jax.experimental import pallas as pl
from jax.experimental.pallas import tpu as pltpu
from jax.experimental.pallas import tpu_sc as plsc

_VOCAB = 1000000
_SEQ_LEN = 200
_EMBED_DIM = 32
_BATCH = 4096

_NC = 2
_NS = 16
_NW = _NC * _NS

_B_FLAT = _BATCH * _SEQ_LEN
_PER_W = _B_FLAT // _NW
_PIECE = 128
_PIECES_PER_CHUNK = 8
_CHUNK = _PIECE * _PIECES_PER_CHUNK
_N_CHUNKS = _PER_W // _CHUNK
_IDX_ROWS_PER_W = _PER_W // _PIECE

_HALF = _EMBED_DIM // 2


def _emb_body(idx_hbm, tok_hbm, pos_hbm, out_hbm, idx_v, rows_v, pos_v, sem):
    wid = lax.axis_index("s") * _NC + lax.axis_index("c")
    idx_row0 = wid * _IDX_ROWS_PER_W
    out_row0 = wid * _PER_W

    pltpu.sync_copy(pos_hbm, pos_v)

    def chunk_body(c, _):
        pltpu.sync_copy(
            idx_hbm.at[pl.ds(idx_row0 + c * _PIECES_PER_CHUNK, _PIECES_PER_CHUNK)],
            idx_v,
        )
        copies = []
        for j in range(_PIECES_PER_CHUNK):
            copies.append(
                pltpu.async_copy(
                    tok_hbm.at[idx_v.at[j]],
                    rows_v.at[pl.ds(j * _PIECE, _PIECE)],
                    sem,
                )
            )
        for cp in copies:
            cp.wait()

        p0 = lax.rem(c * _CHUNK, _SEQ_LEN)

        def add_body(r, _):
            q = lax.rem(p0 + r, _SEQ_LEN) * 2
            plsc.addupdate(rows_v.at[r, pl.ds(0, _HALF)], pos_v[q])
            plsc.addupdate(rows_v.at[r, pl.ds(_HALF, _HALF)], pos_v[q + 1])
            return 0

        lax.fori_loop(0, _CHUNK, add_body, 0)

        pltpu.sync_copy(rows_v, out_hbm.at[pl.ds(out_row0 + c * _CHUNK, _CHUNK)])
        return 0

    lax.fori_loop(0, _N_CHUNKS, chunk_body, 0)


_emb = functools.partial(
    pl.kernel,
    out_type=jax.ShapeDtypeStruct((_B_FLAT, _EMBED_DIM), jnp.float32),
    mesh=plsc.VectorSubcoreMesh(core_axis_name="c", subcore_axis_name="s"),
    scratch_types=[
        pltpu.VMEM((_PIECES_PER_CHUNK, _PIECE), jnp.int32),
        pltpu.VMEM((_CHUNK, _EMBED_DIM), jnp.float32),
        pltpu.VMEM((2 * _SEQ_LEN, _HALF), jnp.float32),
        pltpu.SemaphoreType.DMA,
    ],
)(_emb_body)


@jax.jit
def kernel(inputs, token_table, position_table):
    idx2d = inputs.reshape(_B_FLAT // _PIECE, _PIECE).astype(jnp.int32)
    pos2d = position_table.reshape(2 * _SEQ_LEN, _HALF)
    out = _emb(idx2d, token_table, pos2d)
    return out.reshape(_BATCH, _SEQ_LEN, _EMBED_DIM)


# speedup vs baseline: 1.2287x; 1.2287x over previous
import functools

import jax
import jax.numpy as jnp
from jax import lax
from jax.experimental import pallas as pl
from jax.experimental.pallas import tpu as pltpu
from jax.experimental.pallas import tpu_sc as plsc

_VOCAB = 1000000
_SEQ_LEN = 200
_EMBED_DIM = 32
_BATCH = 4096

_NC = 2
_NS = 16
_NW = _NC * _NS

_B_FLAT = _BATCH * _SEQ_LEN
_PER_W = _B_FLAT // _NW
_PIECE = 128
_PIECES_PER_CHUNK = 8
_CHUNK = _PIECE * _PIECES_PER_CHUNK
_N_CHUNKS = _PER_W // _CHUNK
_IDX_ROWS_PER_W = _PER_W // _PIECE

_HALF = _EMBED_DIM // 2


def _emb_body(idx_hbm, tok_hbm, pos_hbm, out_hbm, idx_v, rows_v, pos_v, sem):
    wid = lax.axis_index("s") * _NC + lax.axis_index("c")
    idx_row0 = wid * _IDX_ROWS_PER_W
    out_row0 = wid * _PER_W

    pltpu.sync_copy(pos_hbm, pos_v)

    def chunk_body(c, _):
        pltpu.sync_copy(
            idx_hbm.at[pl.ds(idx_row0 + c * _PIECES_PER_CHUNK, _PIECES_PER_CHUNK)],
            idx_v,
        )
        copies = []
        for j in range(_PIECES_PER_CHUNK):
            copies.append(
                pltpu.async_copy(
                    tok_hbm.at[idx_v.at[j]],
                    rows_v.at[pl.ds(j * _PIECE, _PIECE)],
                    sem,
                )
            )
        for cp in copies:
            cp.wait()

        p0 = lax.rem(c * _CHUNK, _SEQ_LEN)

        def add_body(r, _):
            q = lax.rem(p0 + r, _SEQ_LEN) * 2
            plsc.addupdate(rows_v.at[r, pl.ds(0, _HALF)], pos_v[q])
            plsc.addupdate(rows_v.at[r, pl.ds(_HALF, _HALF)], pos_v[q + 1])
            return 0

        lax.fori_loop(0, _CHUNK, add_body, 0)

        pltpu.sync_copy(rows_v, out_hbm.at[pl.ds(out_row0 + c * _CHUNK, _CHUNK)])
        return 0

    lax.fori_loop(0, _N_CHUNKS, chunk_body, 0)


_emb = functools.partial(
    pl.kernel,
    out_type=jax.ShapeDtypeStruct((_B_FLAT, _EMBED_DIM), jnp.float32),
    mesh=plsc.VectorSubcoreMesh(core_axis_name="c", subcore_axis_name="s"),
    scratch_types=[
        pltpu.VMEM((_PIECES_PER_CHUNK, _PIECE), jnp.int32),
        pltpu.VMEM((_CHUNK, _EMBED_DIM), jnp.float32),
        pltpu.VMEM((2 * _SEQ_LEN, _HALF), jnp.float32),
        pltpu.SemaphoreType.DMA,
    ],
    compiler_params=pltpu.CompilerParams(use_tc_tiling_on_sc=False),
)(_emb_body)


@jax.jit
def kernel(inputs, token_table, position_table):
    idx2d = inputs.reshape(_B_FLAT // _PIECE, _PIECE).astype(jnp.int32)
    pos2d = position_table.reshape(2 * _SEQ_LEN, _HALF)
    out = _emb(idx2d, token_table, pos2d)
    return out.reshape(_BATCH, _SEQ_LEN, _EMBED_DIM)


# speedup vs baseline: 1.4872x; 1.2104x over previous
import functools

import jax
import jax.numpy as jnp
from jax import lax
from jax.experimental import pallas as pl
from jax.experimental.pallas import tpu as pltpu
from jax.experimental.pallas import tpu_sc as plsc

_VOCAB = 1000000
_SEQ_LEN = 200
_EMBED_DIM = 32
_BATCH = 4096

_NC = 2
_NS = 16
_NW = _NC * _NS

_B_FLAT = _BATCH * _SEQ_LEN
_PER_W = _B_FLAT // _NW
_PIECE = 100
_PIECES = 8
_CHUNK = _PIECE * _PIECES
_N_CHUNKS = _PER_W // _CHUNK
_SEQ_PER_CHUNK = _CHUNK // _SEQ_LEN
_IDX_ROWS_PER_W = _PER_W // _PIECE

_HALF = _EMBED_DIM // 2


def _emb_body(idx_hbm, tok_hbm, pos_hbm, out_hbm,
              idx0, idx1, g0, g1, o0, o1, pos_v,
              gsem0, gsem1, osem0, osem1):
    idxs = (idx0, idx1)
    gs = (g0, g1)
    outs = (o0, o1)
    gsems = (gsem0, gsem1)
    osems = (osem0, osem1)

    wid = lax.axis_index("s") * _NC + lax.axis_index("c")
    idx_row0 = wid * _IDX_ROWS_PER_W
    out_row0 = wid * _PER_W

    pltpu.sync_copy(pos_hbm, pos_v)

    def gather_descs(b):
        return [
            pltpu.make_async_copy(
                tok_hbm.at[idxs[b].at[j]],
                gs[b].at[pl.ds(j * _PIECE, _PIECE)],
                gsems[b],
            )
            for j in range(_PIECES)
        ]

    def stage_and_fire(c, b):
        pltpu.sync_copy(idx_hbm.at[pl.ds(idx_row0 + c * _PIECES, _PIECES)], idxs[b])
        for d in gather_descs(b):
            d.start()

    def drain_gathers(b):
        for d in gather_descs(b):
            d.wait()

    def out_desc(c, b):
        return pltpu.make_async_copy(
            outs[b], out_hbm.at[pl.ds(out_row0 + c * _CHUNK, _CHUNK)], osems[b]
        )

    def add_pos(b):
        @plsc.parallel_loop(0, _SEQ_LEN, unroll=2)
        def _(r):
            q = r * 2
            pv0 = pos_v[q]
            pv1 = pos_v[q + 1]
            for s in range(_SEQ_PER_CHUNK):
                row = s * _SEQ_LEN + r
                outs[b][row, pl.ds(0, _HALF)] = gs[b][row, pl.ds(0, _HALF)] + pv0
                outs[b][row, pl.ds(_HALF, _HALF)] = gs[b][row, pl.ds(_HALF, _HALF)] + pv1

    stage_and_fire(0, 0)
    stage_and_fire(1, 1)

    for b in range(2):
        drain_gathers(b)
        add_pos(b)
        out_desc(b, b).start()
        stage_and_fire(b + 2, b)

    @pl.loop(2, _N_CHUNKS - 2, step=2)
    def _(s):
        for b in range(2):
            c = s + b
            drain_gathers(b)
            out_desc(c - 2, b).wait()
            add_pos(b)
            out_desc(c, b).start()
            stage_and_fire(c + 2, b)

    for b in range(2):
        c = _N_CHUNKS - 2 + b
        drain_gathers(b)
        out_desc(c - 2, b).wait()
        add_pos(b)
        out_desc(c, b).start()

    for b in range(2):
        out_desc(_N_CHUNKS - 2 + b, b).wait()


_emb = functools.partial(
    pl.kernel,
    out_type=jax.ShapeDtypeStruct((_B_FLAT, _EMBED_DIM), jnp.float32),
    mesh=plsc.VectorSubcoreMesh(core_axis_name="c", subcore_axis_name="s"),
    scratch_types=[
        pltpu.VMEM((_PIECES, _PIECE), jnp.int32),
        pltpu.VMEM((_PIECES, _PIECE), jnp.int32),
        pltpu.VMEM((_CHUNK, _EMBED_DIM), jnp.float32),
        pltpu.VMEM((_CHUNK, _EMBED_DIM), jnp.float32),
        pltpu.VMEM((_CHUNK, _EMBED_DIM), jnp.float32),
        pltpu.VMEM((_CHUNK, _EMBED_DIM), jnp.float32),
        pltpu.VMEM((2 * _SEQ_LEN, _HALF), jnp.float32),
        pltpu.SemaphoreType.DMA,
        pltpu.SemaphoreType.DMA,
        pltpu.SemaphoreType.DMA,
        pltpu.SemaphoreType.DMA,
    ],
    compiler_params=pltpu.CompilerParams(use_tc_tiling_on_sc=False),
)(_emb_body)


@jax.jit
def kernel(inputs, token_table, position_table):
    idx2d = inputs.reshape(_B_FLAT // _PIECE, _PIECE).astype(jnp.int32)
    pos2d = position_table.reshape(2 * _SEQ_LEN, _HALF)
    out = _emb(idx2d, token_table, pos2d)
    return out.reshape(_BATCH, _SEQ_LEN, _EMBED_DIM)


# speedup vs baseline: 1.5716x; 1.0568x over previous
import functools

import jax
import jax.numpy as jnp
from jax import lax
from jax.experimental import pallas as pl
from jax.experimental.pallas import tpu as pltpu
from jax.experimental.pallas import tpu_sc as plsc

_VOCAB = 1000000
_SEQ_LEN = 200
_EMBED_DIM = 32
_BATCH = 4096

_NC = 2
_NS = 16
_NW = _NC * _NS

_BBLK = _BATCH // _NW
_S_PER_CHUNK = 4
_CHUNK_ROWS = _S_PER_CHUNK * _BBLK
_N_CHUNKS = _SEQ_LEN // _S_PER_CHUNK

_ST = _SEQ_LEN // 8
_BT = _BATCH // 128

_EG = _EMBED_DIM // 8
_TILE = 8 * 128
_S_STRIDE = _EG * _BT * _TILE
_G_STRIDE = _BT * _TILE
_OUT_WORDS = _SEQ_LEN * _S_STRIDE

_HALF = 16


def _emb_body(idx_hbm, tok_hbm, pos_hbm, out_hbm,
              idx0, idx1, g0, g1, t0, t1, pos_v,
              gsem0, gsem1, osem0, osem1):
    idxs = (idx0, idx1)
    gs = (g0, g1)
    tiles = (t0, t1)
    gsems = (gsem0, gsem1)
    osems = (osem0, osem1)

    wid = lax.axis_index("s") * _NC + lax.axis_index("c")

    pltpu.sync_copy(pos_hbm, pos_v)

    lane = lax.iota(jnp.int32, 16)
    pat0 = ((lane >> 3) << 10) + ((lane & 7) << 7)
    pat1 = pat0 + 2 * _TILE

    def idx_and_gathers(c, b):
        pltpu.sync_copy(
            idx_hbm.at[c // 2, wid, pl.ds((c % 2) * _S_PER_CHUNK, _S_PER_CHUNK)],
            idxs[b],
        )
        for d in gather_descs(c, b):
            d.start()

    def gather_descs(c, b):
        return [
            pltpu.make_async_copy(
                tok_hbm.at[idxs[b].at[j]],
                gs[b].at[pl.ds(j * _BBLK, _BBLK)],
                gsems[b],
            )
            for j in range(_S_PER_CHUNK)
        ]

    def drain_gathers(c, b):
        for d in gather_descs(c, b):
            d.wait()

    def transpose_add(c, b):
        for j in range(_S_PER_CHUNK):
            s = c * _S_PER_CHUNK + j
            pv0 = pos_v[s, pl.ds(0, _HALF)]
            pv1 = pos_v[s, pl.ds(_HALF, _HALF)]

            @plsc.parallel_loop(0, _BBLK, unroll=4)
            def _(bl):
                r = j * _BBLK + bl
                v0 = gs[b][r, pl.ds(0, _HALF)] + pv0
                v1 = gs[b][r, pl.ds(_HALF, _HALF)] + pv1
                plsc.store_scatter(tiles[b].at[j], [pat0 + bl], v0)
                plsc.store_scatter(tiles[b].at[j], [pat1 + bl], v1)

    def out_descs(c, b):
        ds_ = []
        for j in range(_S_PER_CHUNK):
            s = c * _S_PER_CHUNK + j
            for g in range(_EG):
                ds_.append(
                    pltpu.make_async_copy(
                        tiles[b].at[j, pl.ds(g * _TILE, _TILE)],
                        out_hbm.at[
                            pl.ds(s * _S_STRIDE + g * _G_STRIDE + wid * _TILE, _TILE)
                        ],
                        osems[b],
                    )
                )
        return ds_

    idx_and_gathers(0, 0)
    idx_and_gathers(1, 1)

    for b in range(2):
        drain_gathers(b, b)
        transpose_add(b, b)
        for d in out_descs(b, b):
            d.start()
        idx_and_gathers(b + 2, b)

    @pl.loop(2, _N_CHUNKS - 2, step=2)
    def _(sc):
        for b in range(2):
            c = sc + b
            drain_gathers(c, b)
            for d in out_descs(c - 2, b):
                d.wait()
            transpose_add(c, b)
            for d in out_descs(c, b):
                d.start()
            idx_and_gathers(c + 2, b)

    for b in range(2):
        c = _N_CHUNKS - 2 + b
        drain_gathers(c, b)
        for d in out_descs(c - 2, b):
            d.wait()
        transpose_add(c, b)
        for d in out_descs(c, b):
            d.start()

    for b in range(2):
        for d in out_descs(_N_CHUNKS - 2 + b, b):
            d.wait()


_emb = functools.partial(
    pl.kernel,
    out_type=jax.ShapeDtypeStruct((_OUT_WORDS,), jnp.float32),
    mesh=plsc.VectorSubcoreMesh(core_axis_name="c", subcore_axis_name="s"),
    scratch_types=[
        pltpu.VMEM((_S_PER_CHUNK, _BBLK), jnp.int32),
        pltpu.VMEM((_S_PER_CHUNK, _BBLK), jnp.int32),
        pltpu.VMEM((_CHUNK_ROWS, _EMBED_DIM), jnp.float32),
        pltpu.VMEM((_CHUNK_ROWS, _EMBED_DIM), jnp.float32),
        pltpu.VMEM((_S_PER_CHUNK, _EG * _TILE), jnp.float32),
        pltpu.VMEM((_S_PER_CHUNK, _EG * _TILE), jnp.float32),
        pltpu.VMEM((_SEQ_LEN, _EMBED_DIM), jnp.float32),
        pltpu.SemaphoreType.DMA,
        pltpu.SemaphoreType.DMA,
        pltpu.SemaphoreType.DMA,
        pltpu.SemaphoreType.DMA,
    ],
    compiler_params=pltpu.CompilerParams(
        use_tc_tiling_on_sc=False, needs_layout_passes=False
    ),
)(_emb_body)


@jax.jit
def kernel(inputs, token_table, position_table):
    idx4d = (
        inputs.astype(jnp.int32).T
        .reshape(_ST, 8, _BT, 128)
        .transpose(0, 2, 1, 3)
    )
    flat = _emb(idx4d, token_table, position_table)
    out = (
        flat.reshape(_SEQ_LEN, _EG, _BT, 8, 128)
        .transpose(2, 4, 0, 1, 3)
        .reshape(_BATCH, _SEQ_LEN, _EMBED_DIM)
    )
    return out


# speedup vs baseline: 1.5790x; 1.0047x over previous
import functools

import jax
import jax.numpy as jnp
from jax import lax
from jax.experimental import pallas as pl
from jax.experimental.pallas import tpu as pltpu
from jax.experimental.pallas import tpu_sc as plsc

_VOCAB = 1000000
_SEQ_LEN = 200
_EMBED_DIM = 32
_BATCH = 4096

_NC = 2
_NS = 16
_NW = _NC * _NS

_BBLK = _BATCH // _NW
_S_PER_CHUNK = 4
_CHUNK_ROWS = _S_PER_CHUNK * _BBLK
_N_CHUNKS = _SEQ_LEN // _S_PER_CHUNK

_ST = _SEQ_LEN // 8
_BT = _BATCH // 128

_EG = _EMBED_DIM // 8
_TILE = 8 * 128
_S_STRIDE = _EG * _BT * _TILE
_G_STRIDE = _BT * _TILE
_OUT_WORDS = _SEQ_LEN * _S_STRIDE

_HALF = 16


def _emb_body(idx_hbm, tok_hbm, pos_hbm, out_hbm,
              idx0, idx1, g0, g1, t0, t1, pos_v,
              gsem0, gsem1, osem0, osem1):
    idxs = (idx0, idx1)
    gs = (g0, g1)
    tiles = (t0, t1)
    gsems = (gsem0, gsem1)
    osems = (osem0, osem1)

    wid = lax.axis_index("s") * _NC + lax.axis_index("c")

    pltpu.sync_copy(pos_hbm, pos_v)

    lane = lax.iota(jnp.int32, 16)
    grp0 = lane >> 3
    grp1 = grp0 + 2
    wrd = (lane & 7) << 7

    def idx_and_gathers(c, b):
        pltpu.sync_copy(
            idx_hbm.at[c // 2, wid, pl.ds((c % 2) * _S_PER_CHUNK, _S_PER_CHUNK)],
            idxs[b],
        )
        for d in gather_descs(c, b):
            d.start()

    def gather_descs(c, b):
        return [
            pltpu.make_async_copy(
                tok_hbm.at[idxs[b].at[j]],
                gs[b].at[pl.ds(j * _BBLK, _BBLK)],
                gsems[b],
            )
            for j in range(_S_PER_CHUNK)
        ]

    def drain_gathers(c, b):
        for d in gather_descs(c, b):
            d.wait()

    def transpose_add(c, b):
        for j in range(_S_PER_CHUNK):
            s = c * _S_PER_CHUNK + j
            pv0 = pos_v[s, pl.ds(0, _HALF)]
            pv1 = pos_v[s, pl.ds(_HALF, _HALF)]

            @plsc.parallel_loop(0, _BBLK, unroll=4)
            def _(bl):
                r = j * _BBLK + bl
                v0 = gs[b][r, pl.ds(0, _HALF)] + pv0
                v1 = gs[b][r, pl.ds(_HALF, _HALF)] + pv1
                plsc.store_scatter(tiles[b].at[j], [grp0, wrd + bl], v0)
                plsc.store_scatter(tiles[b].at[j], [grp1, wrd + bl], v1)

    def out_descs(c, b):
        return [
            pltpu.make_async_copy(
                tiles[b],
                out_hbm.at[
                    pl.ds(c * _S_PER_CHUNK, _S_PER_CHUNK),
                    slice(None),
                    pl.ds(wid * _TILE, _TILE),
                ],
                osems[b],
            )
        ]

    idx_and_gathers(0, 0)
    idx_and_gathers(1, 1)

    for b in range(2):
        drain_gathers(b, b)
        transpose_add(b, b)
        for d in out_descs(b, b):
            d.start()
        idx_and_gathers(b + 2, b)

    @pl.loop(2, _N_CHUNKS - 2, step=2)
    def _(sc):
        for b in range(2):
            c = sc + b
            drain_gathers(c, b)
            for d in out_descs(c - 2, b):
                d.wait()
            transpose_add(c, b)
            for d in out_descs(c, b):
                d.start()
            idx_and_gathers(c + 2, b)

    for b in range(2):
        c = _N_CHUNKS - 2 + b
        drain_gathers(c, b)
        for d in out_descs(c - 2, b):
            d.wait()
        transpose_add(c, b)
        for d in out_descs(c, b):
            d.start()

    for b in range(2):
        for d in out_descs(_N_CHUNKS - 2 + b, b):
            d.wait()


_emb = functools.partial(
    pl.kernel,
    out_type=jax.ShapeDtypeStruct((_SEQ_LEN, _EG, _BT * _TILE), jnp.float32),
    mesh=plsc.VectorSubcoreMesh(core_axis_name="c", subcore_axis_name="s"),
    scratch_types=[
        pltpu.VMEM((_S_PER_CHUNK, _BBLK), jnp.int32),
        pltpu.VMEM((_S_PER_CHUNK, _BBLK), jnp.int32),
        pltpu.VMEM((_CHUNK_ROWS, _EMBED_DIM), jnp.float32),
        pltpu.VMEM((_CHUNK_ROWS, _EMBED_DIM), jnp.float32),
        pltpu.VMEM((_S_PER_CHUNK, _EG, _TILE), jnp.float32),
        pltpu.VMEM((_S_PER_CHUNK, _EG, _TILE), jnp.float32),
        pltpu.VMEM((_SEQ_LEN, _EMBED_DIM), jnp.float32),
        pltpu.SemaphoreType.DMA,
        pltpu.SemaphoreType.DMA,
        pltpu.SemaphoreType.DMA,
        pltpu.SemaphoreType.DMA,
    ],
    compiler_params=pltpu.CompilerParams(
        use_tc_tiling_on_sc=False, needs_layout_passes=False
    ),
)(_emb_body)


@jax.jit
def kernel(inputs, token_table, position_table):
    idx4d = (
        inputs.astype(jnp.int32).T
        .reshape(_ST, 8, _BT, 128)
        .transpose(0, 2, 1, 3)
    )
    flat = _emb(idx4d, token_table, position_table)
    out = (
        flat.reshape(_SEQ_LEN, _EG, _BT, 8, 128)
        .transpose(2, 4, 0, 1, 3)
        .reshape(_BATCH, _SEQ_LEN, _EMBED_DIM)
    )
    return out


# speedup vs baseline: 1.6577x; 1.0498x over previous
import functools

import jax
import jax.numpy as jnp
from jax import lax
from jax.experimental import pallas as pl
from jax.experimental.pallas import tpu as pltpu
from jax.experimental.pallas import tpu_sc as plsc

_VOCAB = 1000000
_SEQ_LEN = 200
_EMBED_DIM = 32
_BATCH = 4096

_NC = 2
_NS = 16
_NW = _NC * _NS

_BBLK = _BATCH // _NW
_S_PER_CHUNK = 4
_CHUNK_ROWS = _S_PER_CHUNK * _BBLK
_N_CHUNKS = _SEQ_LEN // _S_PER_CHUNK

_ST = _SEQ_LEN // 8
_BT = _BATCH // 128

_EG = _EMBED_DIM // 8
_TILE = 8 * 128
_S_STRIDE = _EG * _BT * _TILE
_G_STRIDE = _BT * _TILE
_OUT_WORDS = _SEQ_LEN * _S_STRIDE

_HALF = 16


def _emb_body(idx_hbm, tok_hbm, pos_hbm, out_hbm,
              idx0, idx1, g0, g1, t0, t1, pos_v,
              gsem0, gsem1, osem0, osem1):
    idxs = (idx0, idx1)
    gs = (g0, g1)
    tiles = (t0, t1)
    gsems = (gsem0, gsem1)
    osems = (osem0, osem1)

    wid = lax.axis_index("s") * _NC + lax.axis_index("c")

    pltpu.sync_copy(pos_hbm, pos_v)

    lane = lax.iota(jnp.int32, 16)

    def idx_and_gathers(c, b):
        pltpu.sync_copy(
            idx_hbm.at[c // 2, wid, pl.ds((c % 2) * _S_PER_CHUNK, _S_PER_CHUNK)],
            idxs[b],
        )
        for d in gather_descs(c, b):
            d.start()

    def gather_descs(c, b):
        return [
            pltpu.make_async_copy(
                tok_hbm.at[idxs[b].at[j]],
                gs[b].at[pl.ds(j * _BBLK, _BBLK)],
                gsems[b],
            )
            for j in range(_S_PER_CHUNK)
        ]

    def drain_gathers(c, b):
        for d in gather_descs(c, b):
            d.wait()

    def transpose_add(c, b):
        for j in range(_S_PER_CHUNK):
            s = c * _S_PER_CHUNK + j

            sv = lax.broadcast_in_dim(s, (16,), ())

            @plsc.parallel_loop(0, _EMBED_DIM, unroll=2)
            def _(e):
                colv = lax.broadcast_in_dim(e, (16,), ())
                pv = plsc.load_gather(pos_v, [sv, colv])
                e8 = e >> 3
                ew = (e & 7) << 7
                for b0 in range(_BBLK // 16):
                    rowv = lane + (j * _BBLK + b0 * 16)
                    v = plsc.load_gather(gs[b], [rowv, colv]) + pv
                    tiles[b][j, e8, pl.ds(ew + b0 * 16, 16)] = v

    def out_descs(c, b):
        return [
            pltpu.make_async_copy(
                tiles[b],
                out_hbm.at[
                    pl.ds(c * _S_PER_CHUNK, _S_PER_CHUNK),
                    slice(None),
                    pl.ds(wid * _TILE, _TILE),
                ],
                osems[b],
            )
        ]

    idx_and_gathers(0, 0)
    idx_and_gathers(1, 1)

    for b in range(2):
        drain_gathers(b, b)
        transpose_add(b, b)
        for d in out_descs(b, b):
            d.start()
        idx_and_gathers(b + 2, b)

    @pl.loop(2, _N_CHUNKS - 2, step=2)
    def _(sc):
        for b in range(2):
            c = sc + b
            drain_gathers(c, b)
            for d in out_descs(c - 2, b):
                d.wait()
            transpose_add(c, b)
            for d in out_descs(c, b):
                d.start()
            idx_and_gathers(c + 2, b)

    for b in range(2):
        c = _N_CHUNKS - 2 + b
        drain_gathers(c, b)
        for d in out_descs(c - 2, b):
            d.wait()
        transpose_add(c, b)
        for d in out_descs(c, b):
            d.start()

    for b in range(2):
        for d in out_descs(_N_CHUNKS - 2 + b, b):
            d.wait()


_emb = functools.partial(
    pl.kernel,
    out_type=jax.ShapeDtypeStruct((_SEQ_LEN, _EG, _BT * _TILE), jnp.float32),
    mesh=plsc.VectorSubcoreMesh(core_axis_name="c", subcore_axis_name="s"),
    scratch_types=[
        pltpu.VMEM((_S_PER_CHUNK, _BBLK), jnp.int32),
        pltpu.VMEM((_S_PER_CHUNK, _BBLK), jnp.int32),
        pltpu.VMEM((_CHUNK_ROWS, _EMBED_DIM), jnp.float32),
        pltpu.VMEM((_CHUNK_ROWS, _EMBED_DIM), jnp.float32),
        pltpu.VMEM((_S_PER_CHUNK, _EG, _TILE), jnp.float32),
        pltpu.VMEM((_S_PER_CHUNK, _EG, _TILE), jnp.float32),
        pltpu.VMEM((_SEQ_LEN, _EMBED_DIM), jnp.float32),
        pltpu.SemaphoreType.DMA,
        pltpu.SemaphoreType.DMA,
        pltpu.SemaphoreType.DMA,
        pltpu.SemaphoreType.DMA,
    ],
    compiler_params=pltpu.CompilerParams(
        use_tc_tiling_on_sc=False, needs_layout_passes=False
    ),
)(_emb_body)


@jax.jit
def kernel(inputs, token_table, position_table):
    idx4d = (
        inputs.astype(jnp.int32).T
        .reshape(_ST, 8, _BT, 128)
        .transpose(0, 2, 1, 3)
    )
    flat = _emb(idx4d, token_table, position_table)
    out = (
        flat.reshape(_SEQ_LEN, _EG, _BT, 8, 128)
        .transpose(2, 4, 0, 1, 3)
        .reshape(_BATCH, _SEQ_LEN, _EMBED_DIM)
    )
    return out


# speedup vs baseline: 2.4225x; 1.4614x over previous
import functools

import jax
import jax.numpy as jnp
from jax import lax
from jax.experimental import pallas as pl
from jax.experimental.pallas import tpu as pltpu
from jax.experimental.pallas import tpu_sc as plsc

_VOCAB = 1000000
_SEQ_LEN = 200
_EMBED_DIM = 32
_BATCH = 4096

_NC = 2
_NS = 16
_NW = _NC * _NS

_BBLK = _BATCH // _NW
_S_PER_CHUNK = 4
_CHUNK_ROWS = _S_PER_CHUNK * _BBLK
_N_CHUNKS = _SEQ_LEN // _S_PER_CHUNK

_ST = _SEQ_LEN // 8
_BT = _BATCH // 128

_EG = _EMBED_DIM // 8
_TILE = 8 * 128
_PITCH = 129

_HALF = 16


def _emb_body(idx_hbm, tok_hbm, pos_hbm, out_hbm,
              idx0, idx1, g0, g1, t0, t1, pos_v,
              gsem0, gsem1, osem0, osem1):
    idxs = (idx0, idx1)
    gs = (g0, g1)
    tiles = (t0, t1)
    gsems = (gsem0, gsem1)
    osems = (osem0, osem1)

    wid = lax.axis_index("s") * _NC + lax.axis_index("c")

    pltpu.sync_copy(pos_hbm, pos_v)

    lane = lax.iota(jnp.int32, 16)
    gv0 = lane >> 3
    gv1 = gv0 + 2
    e8v = lane & 7

    def idx_and_gathers(c, b):
        pltpu.sync_copy(
            idx_hbm.at[c // 2, wid, pl.ds((c % 2) * _S_PER_CHUNK, _S_PER_CHUNK)],
            idxs[b],
        )
        for d in gather_descs(c, b):
            d.start()

    def gather_descs(c, b):
        return [
            pltpu.make_async_copy(
                tok_hbm.at[idxs[b].at[j]],
                gs[b].at[pl.ds(j * _BBLK, _BBLK)],
                gsems[b],
            )
            for j in range(_S_PER_CHUNK)
        ]

    def drain_gathers(c, b):
        for d in gather_descs(c, b):
            d.wait()

    def transpose_add(c, b):
        for j in range(_S_PER_CHUNK):
            s = c * _S_PER_CHUNK + j
            pv0 = pos_v[s, pl.ds(0, _HALF)]
            pv1 = pos_v[s, pl.ds(_HALF, _HALF)]

            @plsc.parallel_loop(0, _BBLK, unroll=4)
            def _(bl):
                r = j * _BBLK + bl
                wv = lax.broadcast_in_dim(bl, (16,), ())
                v0 = gs[b][r, pl.ds(0, _HALF)] + pv0
                v1 = gs[b][r, pl.ds(_HALF, _HALF)] + pv1
                plsc.store_scatter(tiles[b].at[j], [gv0, e8v, wv], v0)
                plsc.store_scatter(tiles[b].at[j], [gv1, e8v, wv], v1)

    def out_descs(c, b):
        ds_ = []
        for j in range(_S_PER_CHUNK):
            s = c * _S_PER_CHUNK + j
            ds_.append(
                pltpu.make_async_copy(
                    tiles[b].at[j, slice(None), slice(None), pl.ds(0, 128)],
                    out_hbm.at[s, slice(None), wid],
                    osems[b],
                )
            )
        return ds_

    idx_and_gathers(0, 0)
    idx_and_gathers(1, 1)

    for b in range(2):
        drain_gathers(b, b)
        transpose_add(b, b)
        for d in out_descs(b, b):
            d.start()
        idx_and_gathers(b + 2, b)

    @pl.loop(2, _N_CHUNKS - 2, step=2)
    def _(sc):
        for b in range(2):
            c = sc + b
            drain_gathers(c, b)
            for d in out_descs(c - 2, b):
                d.wait()
            transpose_add(c, b)
            for d in out_descs(c, b):
                d.start()
            idx_and_gathers(c + 2, b)

    for b in range(2):
        c = _N_CHUNKS - 2 + b
        drain_gathers(c, b)
        for d in out_descs(c - 2, b):
            d.wait()
        transpose_add(c, b)
        for d in out_descs(c, b):
            d.start()

    for b in range(2):
        for d in out_descs(_N_CHUNKS - 2 + b, b):
            d.wait()


_emb = functools.partial(
    pl.kernel,
    out_type=jax.ShapeDtypeStruct((_SEQ_LEN, _EG, _BT, 8, 128), jnp.float32),
    mesh=plsc.VectorSubcoreMesh(core_axis_name="c", subcore_axis_name="s"),
    scratch_types=[
        pltpu.VMEM((_S_PER_CHUNK, _BBLK), jnp.int32),
        pltpu.VMEM((_S_PER_CHUNK, _BBLK), jnp.int32),
        pltpu.VMEM((_CHUNK_ROWS, _EMBED_DIM), jnp.float32),
        pltpu.VMEM((_CHUNK_ROWS, _EMBED_DIM), jnp.float32),
        pltpu.VMEM((_S_PER_CHUNK, _EG, 8, _PITCH), jnp.float32),
        pltpu.VMEM((_S_PER_CHUNK, _EG, 8, _PITCH), jnp.float32),
        pltpu.VMEM((_SEQ_LEN, _EMBED_DIM), jnp.float32),
        pltpu.SemaphoreType.DMA,
        pltpu.SemaphoreType.DMA,
        pltpu.SemaphoreType.DMA,
        pltpu.SemaphoreType.DMA,
    ],
    compiler_params=pltpu.CompilerParams(
        use_tc_tiling_on_sc=False, needs_layout_passes=False
    ),
)(_emb_body)


@jax.jit
def kernel(inputs, token_table, position_table):
    idx4d = (
        inputs.astype(jnp.int32).T
        .reshape(_ST, 8, _BT, 128)
        .transpose(0, 2, 1, 3)
    )
    flat = _emb(idx4d, token_table, position_table)
    out = (
        flat.reshape(_SEQ_LEN, _EG, _BT, 8, 128)
        .transpose(2, 4, 0, 1, 3)
        .reshape(_BATCH, _SEQ_LEN, _EMBED_DIM)
    )
    return out
